# Initial kernel scaffold; baseline (speedup 1.0000x reference)
import jax
import jax.numpy as jnp
from jax.experimental import pallas as pl


def kernel(predicted_tensor, target_tensor):
    raise NotImplementedError("write your pallas kernel here")


# speedup vs baseline: 7.5069x; 7.5069x over previous
import functools
import math

import jax
import jax.numpy as jnp
from jax import lax
from jax.experimental import pallas as pl

SMOOTH = 0.1
PAD = 0

BR = 256
BC = 6400


def _loss_kernel(tgt_ref, x_ref, o_ref, *, eps, const, nvb):
    i = pl.program_id(0)
    j = pl.program_id(1)

    @pl.when(jnp.logical_and(i == 0, j == 0))
    def _():
        o_ref[...] = jnp.zeros((1, 1), jnp.float32)

    x = x_ref[...]
    t = tgt_ref[...].astype(jnp.int32)
    cols = lax.broadcasted_iota(jnp.int32, (BR, BC), 1) + j * BC
    pts = jnp.sum(jnp.where(cols == t, x, 0.0), axis=1)
    rs = jnp.sum(x, axis=1)
    mask = t[:, 0] != PAD
    part = jnp.sum(jnp.where(mask, -eps * rs + (eps - 0.9) * pts, 0.0))
    first = (j == 0).astype(jnp.float32)
    part = part + first * jnp.sum(jnp.where(mask, const + eps * x[:, 0], 0.0))
    o_ref[...] += part.reshape(1, 1)


def kernel(predicted_tensor, target_tensor):
    B, S, V = predicted_tensor.shape
    N = B * S
    pred = predicted_tensor.reshape(N, V)
    tgt = target_tensor.reshape(N, 1).astype(jnp.int32)

    eps = SMOOTH / (V - 2)
    const = SMOOTH * math.log(eps) + (1.0 - SMOOTH) * math.log(1.0 - SMOOTH)

    nvb = V // BC
    grid = (N // BR, nvb)
    out = pl.pallas_call(
        functools.partial(_loss_kernel, eps=eps, const=float(const), nvb=nvb),
        grid=grid,
        in_specs=[
            pl.BlockSpec((BR, 1), lambda i, j: (i, 0)),
            pl.BlockSpec((BR, BC), lambda i, j: (i, j)),
        ],
        out_specs=pl.BlockSpec((1, 1), lambda i, j: (0, 0)),
        out_shape=jax.ShapeDtypeStruct((1, 1), jnp.float32),
    )(tgt, pred)
    return out[0, 0]


# speedup vs baseline: 8.2374x; 1.0973x over previous
import functools
import math

import jax
import jax.numpy as jnp
from jax import lax
from jax.experimental import pallas as pl

SMOOTH = 0.1
PAD = 0

BR = 256
BC = 16000


def _loss_kernel(tgt_ref, x_ref, o_ref, *, eps, const, nvb):
    i = pl.program_id(0)
    j = pl.program_id(1)

    @pl.when(jnp.logical_and(i == 0, j == 0))
    def _():
        o_ref[...] = jnp.zeros((1, 1), jnp.float32)

    x = x_ref[...]
    t = tgt_ref[...].astype(jnp.int32)
    cols = lax.broadcasted_iota(jnp.int32, (BR, BC), 1) + j * BC
    pts = jnp.sum(jnp.where(cols == t, x, 0.0), axis=1)
    rs = jnp.sum(x, axis=1)
    mask = t[:, 0] != PAD
    part = jnp.sum(jnp.where(mask, -eps * rs + (eps - 0.9) * pts, 0.0))
    first = (j == 0).astype(jnp.float32)
    part = part + first * jnp.sum(jnp.where(mask, const + eps * x[:, 0], 0.0))
    o_ref[...] += part.reshape(1, 1)


def kernel(predicted_tensor, target_tensor):
    B, S, V = predicted_tensor.shape
    N = B * S
    pred = predicted_tensor.reshape(N, V)
    tgt = target_tensor.reshape(N, 1).astype(jnp.int32)

    eps = SMOOTH / (V - 2)
    const = SMOOTH * math.log(eps) + (1.0 - SMOOTH) * math.log(1.0 - SMOOTH)

    nvb = V // BC
    grid = (N // BR, nvb)
    out = pl.pallas_call(
        functools.partial(_loss_kernel, eps=eps, const=float(const), nvb=nvb),
        grid=grid,
        in_specs=[
            pl.BlockSpec((BR, 1), lambda i, j: (i, 0)),
            pl.BlockSpec((BR, BC), lambda i, j: (i, j)),
        ],
        out_specs=pl.BlockSpec((1, 1), lambda i, j: (0, 0)),
        out_shape=jax.ShapeDtypeStruct((1, 1), jnp.float32),
    )(tgt, pred)
    return out[0, 0]


# speedup vs baseline: 8.2527x; 1.0019x over previous
import functools
import math

import jax
import jax.numpy as jnp
from jax import lax
from jax.experimental import pallas as pl

SMOOTH = 0.1
PAD = 0

BR = 128
BC = 32000


def _loss_kernel(tgt_ref, x_ref, o_ref, *, eps, const, nvb):
    i = pl.program_id(0)
    j = pl.program_id(1)

    @pl.when(jnp.logical_and(i == 0, j == 0))
    def _():
        o_ref[...] = jnp.zeros((1, 1), jnp.float32)

    x = x_ref[...]
    t = tgt_ref[...].astype(jnp.int32)
    cols = lax.broadcasted_iota(jnp.int32, (BR, BC), 1) + j * BC
    pts = jnp.sum(jnp.where(cols == t, x, 0.0), axis=1)
    rs = jnp.sum(x, axis=1)
    mask = t[:, 0] != PAD
    part = jnp.sum(jnp.where(mask, -eps * rs + (eps - 0.9) * pts, 0.0))
    first = (j == 0).astype(jnp.float32)
    part = part + first * jnp.sum(jnp.where(mask, const + eps * x[:, 0], 0.0))
    o_ref[...] += part.reshape(1, 1)


def kernel(predicted_tensor, target_tensor):
    B, S, V = predicted_tensor.shape
    N = B * S
    pred = predicted_tensor.reshape(N, V)
    tgt = target_tensor.reshape(N, 1).astype(jnp.int32)

    eps = SMOOTH / (V - 2)
    const = SMOOTH * math.log(eps) + (1.0 - SMOOTH) * math.log(1.0 - SMOOTH)

    nvb = V // BC
    grid = (N // BR, nvb)
    out = pl.pallas_call(
        functools.partial(_loss_kernel, eps=eps, const=float(const), nvb=nvb),
        grid=grid,
        in_specs=[
            pl.BlockSpec((BR, 1), lambda i, j: (i, 0)),
            pl.BlockSpec((BR, BC), lambda i, j: (i, j)),
        ],
        out_specs=pl.BlockSpec((1, 1), lambda i, j: (0, 0)),
        out_shape=jax.ShapeDtypeStruct((1, 1), jnp.float32),
    )(tgt, pred)
    return out[0, 0]
